# baseline (device time: 46273 ns/iter reference)
import jax
import jax.numpy as jnp
from jax import lax
from jax.experimental import pallas as pl
from jax.experimental.pallas import tpu as pltpu

N_DEV = 4
M_PER = 512
D = 512


def kernel(partial, gamma):
    x = partial.reshape(N_DEV * M_PER, D)
    g = gamma.reshape(1, D)

    def body(x_ref, g_ref, out_ref, comm_ref, send_sems, recv_sems):
        my = lax.axis_index("i")
        left = lax.rem(my + N_DEV - 1, N_DEV)
        right = lax.rem(my + 1, N_DEV)

        barrier_sem = pltpu.get_barrier_semaphore()
        for nbr in (left, right):
            pl.semaphore_signal(
                barrier_sem, inc=1,
                device_id=(nbr,), device_id_type=pl.DeviceIdType.MESH,
            )
        pl.semaphore_wait(barrier_sem, 2)

        c0 = lax.rem(my + N_DEV - 1, N_DEV) * M_PER
        comm_ref[0] = x_ref[pl.ds(c0, M_PER), :]

        for s in range(N_DEV - 1):
            rdma = pltpu.make_async_remote_copy(
                src_ref=comm_ref.at[s],
                dst_ref=comm_ref.at[s + 1],
                send_sem=send_sems.at[s],
                recv_sem=recv_sems.at[s + 1],
                device_id=(right,),
                device_id_type=pl.DeviceIdType.MESH,
            )
            rdma.start()
            rdma.wait()
            c = lax.rem(my + 2 * N_DEV - s - 2, N_DEV) * M_PER
            comm_ref[s + 1] += x_ref[pl.ds(c, M_PER), :]

        y = comm_ref[N_DEV - 1]
        ms = jnp.mean(y * y, axis=-1, keepdims=True)
        out_ref[:, :] = y * lax.rsqrt(ms + 1e-6) * g_ref[:, :]

    return pl.pallas_call(
        body,
        out_shape=jax.ShapeDtypeStruct((M_PER, D), jnp.float32),
        in_specs=[
            pl.BlockSpec(memory_space=pltpu.VMEM),
            pl.BlockSpec(memory_space=pltpu.VMEM),
        ],
        out_specs=pl.BlockSpec(memory_space=pltpu.VMEM),
        scratch_shapes=[
            pltpu.VMEM((N_DEV, M_PER, D), jnp.float32),
            pltpu.SemaphoreType.DMA((N_DEV,)),
            pltpu.SemaphoreType.DMA((N_DEV,)),
        ],
        compiler_params=pltpu.CompilerParams(collective_id=0),
    )(x, g)


# device time: 29546 ns/iter; 1.5661x vs baseline; 1.5661x over previous
import jax
import jax.numpy as jnp
from jax import lax
from jax.experimental import pallas as pl
from jax.experimental.pallas import tpu as pltpu

N_DEV = 4
M_PER = 512
D = 512
DH = D // 2


def kernel(partial, gamma):
    x = partial.reshape(N_DEV * M_PER, D)
    g = gamma.reshape(1, D)

    def body(x_ref, g_ref, out_ref,
             comm_a, comm_b, send_a, recv_a, send_b, recv_b):
        my = lax.axis_index("i")
        left = lax.rem(my + N_DEV - 1, N_DEV)
        right = lax.rem(my + 1, N_DEV)

        barrier_sem = pltpu.get_barrier_semaphore()
        for nbr in (left, right):
            pl.semaphore_signal(
                barrier_sem, inc=1,
                device_id=(nbr,), device_id_type=pl.DeviceIdType.MESH,
            )
        pl.semaphore_wait(barrier_sem, 2)

        ca = lax.rem(my + N_DEV - 1, N_DEV) * M_PER
        cb = lax.rem(my + 1, N_DEV) * M_PER
        comm_a[0] = x_ref[pl.ds(ca, M_PER), :DH]
        comm_b[0] = x_ref[pl.ds(cb, M_PER), DH:]

        for s in range(N_DEV - 1):
            rdma_a = pltpu.make_async_remote_copy(
                src_ref=comm_a.at[s],
                dst_ref=comm_a.at[s + 1],
                send_sem=send_a.at[s],
                recv_sem=recv_a.at[s + 1],
                device_id=(right,),
                device_id_type=pl.DeviceIdType.MESH,
            )
            rdma_b = pltpu.make_async_remote_copy(
                src_ref=comm_b.at[s],
                dst_ref=comm_b.at[s + 1],
                send_sem=send_b.at[s],
                recv_sem=recv_b.at[s + 1],
                device_id=(left,),
                device_id_type=pl.DeviceIdType.MESH,
            )
            rdma_a.start()
            rdma_b.start()
            rdma_a.wait()
            rdma_b.wait()
            ca = lax.rem(my + 2 * N_DEV - s - 2, N_DEV) * M_PER
            cb = lax.rem(my + s + 2, N_DEV) * M_PER
            comm_a[s + 1] += x_ref[pl.ds(ca, M_PER), :DH]
            comm_b[s + 1] += x_ref[pl.ds(cb, M_PER), DH:]

        y = jnp.concatenate([comm_a[N_DEV - 1], comm_b[N_DEV - 1]], axis=1)
        ms = jnp.mean(y * y, axis=-1, keepdims=True)
        out_ref[:, :] = y * lax.rsqrt(ms + 1e-6) * g_ref[:, :]

    return pl.pallas_call(
        body,
        out_shape=jax.ShapeDtypeStruct((M_PER, D), jnp.float32),
        in_specs=[
            pl.BlockSpec(memory_space=pltpu.VMEM),
            pl.BlockSpec(memory_space=pltpu.VMEM),
        ],
        out_specs=pl.BlockSpec(memory_space=pltpu.VMEM),
        scratch_shapes=[
            pltpu.VMEM((N_DEV, M_PER, DH), jnp.float32),
            pltpu.VMEM((N_DEV, M_PER, DH), jnp.float32),
            pltpu.SemaphoreType.DMA((N_DEV,)),
            pltpu.SemaphoreType.DMA((N_DEV,)),
            pltpu.SemaphoreType.DMA((N_DEV,)),
            pltpu.SemaphoreType.DMA((N_DEV,)),
        ],
        compiler_params=pltpu.CompilerParams(collective_id=0),
    )(x, g)


# device time: 25903 ns/iter; 1.7864x vs baseline; 1.1406x over previous
import jax
import jax.numpy as jnp
from jax import lax
from jax.experimental import pallas as pl
from jax.experimental.pallas import tpu as pltpu

N_DEV = 4
M_PER = 512
D = 512
DH = D // 2


def kernel(partial, gamma):
    x = partial.reshape(N_DEV * M_PER, D)
    g = gamma.reshape(1, D)

    def body(x_ref, g_ref, out_ref,
             recv_a1, recv_b1, send_a2, send_b2, recv_a2, recv_b2,
             sems_send_a, sems_recv_a, sems_send_b, sems_recv_b):
        my = lax.axis_index("i")
        left = lax.rem(my + N_DEV - 1, N_DEV)
        right = lax.rem(my + 1, N_DEV)
        q = my ^ 1
        r = 3 - my

        barrier_sem = pltpu.get_barrier_semaphore()
        for nbr in (left, right):
            pl.semaphore_signal(
                barrier_sem, inc=1,
                device_id=(nbr,), device_id_type=pl.DeviceIdType.MESH,
            )
        pl.semaphore_wait(barrier_sem, 2)

        def row(c):
            return pl.ds(c * M_PER, M_PER)

        a1 = []
        for k, c in enumerate((3 - q, q)):
            a1.append(pltpu.make_async_remote_copy(
                src_ref=x_ref.at[row(c), pl.ds(0, DH)],
                dst_ref=recv_a1.at[k],
                send_sem=sems_send_a.at[k],
                recv_sem=sems_recv_a.at[k],
                device_id=(q,),
                device_id_type=pl.DeviceIdType.MESH,
            ))
        b1 = []
        for k, c in enumerate((r ^ 1, r)):
            b1.append(pltpu.make_async_remote_copy(
                src_ref=x_ref.at[row(c), pl.ds(DH, DH)],
                dst_ref=recv_b1.at[k],
                send_sem=sems_send_b.at[k],
                recv_sem=sems_recv_b.at[k],
                device_id=(r,),
                device_id_type=pl.DeviceIdType.MESH,
            ))
        a1[0].start()
        b1[0].start()
        a1[1].start()
        b1[1].start()

        a1[0].wait_recv()
        send_a2[:, :] = recv_a1[0] + x_ref[row(3 - my), pl.ds(0, DH)]
        a2 = pltpu.make_async_remote_copy(
            src_ref=send_a2,
            dst_ref=recv_a2,
            send_sem=sems_send_a.at[2],
            recv_sem=sems_recv_a.at[2],
            device_id=(r,),
            device_id_type=pl.DeviceIdType.MESH,
        )
        a2.start()
        b1[0].wait_recv()
        send_b2[:, :] = recv_b1[0] + x_ref[row(q), pl.ds(DH, DH)]
        b2 = pltpu.make_async_remote_copy(
            src_ref=send_b2,
            dst_ref=recv_b2,
            send_sem=sems_send_b.at[2],
            recv_sem=sems_recv_b.at[2],
            device_id=(q,),
            device_id_type=pl.DeviceIdType.MESH,
        )
        b2.start()

        a1[1].wait_recv()
        b1[1].wait_recv()
        a2.wait_recv()
        b2.wait_recv()
        y_a = recv_a1[1] + x_ref[row(my), pl.ds(0, DH)] + recv_a2[:, :]
        y_b = recv_b1[1] + x_ref[row(my), pl.ds(DH, DH)] + recv_b2[:, :]
        y = jnp.concatenate([y_a, y_b], axis=1)
        ms = jnp.mean(y * y, axis=-1, keepdims=True)
        out_ref[:, :] = y * lax.rsqrt(ms + 1e-6) * g_ref[:, :]

        for d in (a1[0], a1[1], b1[0], b1[1], a2, b2):
            d.wait_send()

    return pl.pallas_call(
        body,
        out_shape=jax.ShapeDtypeStruct((M_PER, D), jnp.float32),
        in_specs=[
            pl.BlockSpec(memory_space=pltpu.VMEM),
            pl.BlockSpec(memory_space=pltpu.VMEM),
        ],
        out_specs=pl.BlockSpec(memory_space=pltpu.VMEM),
        scratch_shapes=[
            pltpu.VMEM((2, M_PER, DH), jnp.float32),
            pltpu.VMEM((2, M_PER, DH), jnp.float32),
            pltpu.VMEM((M_PER, DH), jnp.float32),
            pltpu.VMEM((M_PER, DH), jnp.float32),
            pltpu.VMEM((M_PER, DH), jnp.float32),
            pltpu.VMEM((M_PER, DH), jnp.float32),
            pltpu.SemaphoreType.DMA((3,)),
            pltpu.SemaphoreType.DMA((3,)),
            pltpu.SemaphoreType.DMA((3,)),
            pltpu.SemaphoreType.DMA((3,)),
        ],
        compiler_params=pltpu.CompilerParams(collective_id=0),
    )(x, g)
